# Initial kernel scaffold; baseline (speedup 1.0000x reference)
import functools

import jax
import jax.numpy as jnp
from jax import lax
from jax.experimental import pallas as pl
from jax.experimental.pallas import tpu as pltpu
from jax.experimental.pallas import tpu_sc as plsc

_NC = 2
_NS = 16
_NW = _NC * _NS


def _ssp(h):
    return jnp.maximum(h, 0.0) + jnp.log(0.5 * (1.0 + jnp.exp(-jnp.abs(h))))


def _in2f_body(x_ref, win_ref, y_ref):
    y_ref[0] = jnp.dot(x_ref[0], win_ref[...], preferred_element_type=jnp.float32)


def _in2f(x, Win):
    B, A, NIN = x.shape
    NF = Win.shape[1]
    return pl.pallas_call(
        _in2f_body,
        grid=(B,),
        in_specs=[
            pl.BlockSpec((1, A, NIN), lambda b: (b, 0, 0)),
            pl.BlockSpec((NIN, NF), lambda b: (0, 0)),
        ],
        out_specs=pl.BlockSpec((1, A, NF), lambda b: (b, 0, 0)),
        out_shape=jax.ShapeDtypeStruct((B, A, NF), jnp.float32),
    )(x, Win)


def _sc_gather(y2d, nbrs_flat, A, AN):
    P = nbrs_flat.shape[0]
    D = y2d.shape[1]
    rows_w = P // _NW
    K = 80
    steps = rows_w // K
    mesh = plsc.VectorSubcoreMesh(core_axis_name="c", subcore_axis_name="s")

    @functools.partial(
        pl.kernel,
        out_type=jax.ShapeDtypeStruct((P, D), jnp.float32),
        mesh=mesh,
        scratch_types=[
            pltpu.VMEM((K,), jnp.int32),
            pltpu.VMEM((K, D), jnp.float32),
            pltpu.SemaphoreType.DMA,
        ],
    )
    def k(y_hbm, nbr_hbm, out_hbm, idx_v, rows_v, sem):
        wid = lax.axis_index("s") * _NC + lax.axis_index("c")
        base = wid * rows_w

        def body(j, carry):
            off = base + j * K
            pltpu.sync_copy(nbr_hbm.at[pl.ds(off, K)], idx_v)
            for g in range(K // 16):
                v = idx_v[pl.ds(g * 16, 16)]
                pos = off + g * 16 + lax.iota(jnp.int32, (16,))
                idx_v[pl.ds(g * 16, 16)] = v + (pos // AN) * A
            pltpu.async_copy(y_hbm.at[idx_v], rows_v, sem).wait()
            pltpu.sync_copy(rows_v, out_hbm.at[pl.ds(off, K)])
            return carry

        lax.fori_loop(0, steps, body, 0)

    return k(y2d, nbrs_flat)


def _fr_body(f_ref, yg_ref, m_ref, w1_ref, b1_ref, w2_ref, b2_ref, o_ref):
    TA, NBH, NG = f_ref.shape[1], f_ref.shape[2], f_ref.shape[3]
    NF = w2_ref.shape[1]
    f = f_ref[0].reshape(TA * NBH, NG)
    h = jnp.dot(f, w1_ref[...], preferred_element_type=jnp.float32) + b1_ref[0]
    h = _ssp(h)
    w = jnp.dot(h, w2_ref[...], preferred_element_type=jnp.float32) + b2_ref[0]
    p = w * yg_ref[0] * m_ref[0]
    o_ref[0] = jnp.sum(p.reshape(TA, NBH, NF), axis=1)


def _filter_reduce(f_ij, yg3, mask3, W1, b1, W2, b2, TA=40):
    B, A, NBH, NG = f_ij.shape
    NF = W2.shape[1]
    return pl.pallas_call(
        _fr_body,
        grid=(B, A // TA),
        in_specs=[
            pl.BlockSpec((1, TA, NBH, NG), lambda b, i: (b, i, 0, 0)),
            pl.BlockSpec((1, TA * NBH, NF), lambda b, i: (b, i, 0)),
            pl.BlockSpec((1, TA * NBH, 1), lambda b, i: (b, i, 0)),
            pl.BlockSpec((NG, NF), lambda b, i: (0, 0)),
            pl.BlockSpec((1, NF), lambda b, i: (0, 0)),
            pl.BlockSpec((NF, NF), lambda b, i: (0, 0)),
            pl.BlockSpec((1, NF), lambda b, i: (0, 0)),
        ],
        out_specs=pl.BlockSpec((1, TA, NF), lambda b, i: (b, i, 0)),
        out_shape=jax.ShapeDtypeStruct((B, A, NF), jnp.float32),
    )(f_ij, yg3, mask3, W1, b1.reshape(1, NF), W2, b2.reshape(1, NF))


def kernel(x, r_ij, neighbors, pairwise_mask, f_ij, Win, W1, b1, W2, b2):
    B, A, NBH = neighbors.shape
    NF = Win.shape[1]
    y = _in2f(x, Win)
    yg = _sc_gather(y.reshape(B * A, NF), neighbors.reshape(B * A * NBH), A, A * NBH)
    yg3 = yg.reshape(B, A * NBH, NF)
    mask3 = pairwise_mask.reshape(B, A * NBH, 1)
    return _filter_reduce(f_ij, yg3, mask3, W1, b1, W2, b2)


# speedup vs baseline: 8.1483x; 8.1483x over previous
import functools

import jax
import jax.numpy as jnp
from jax import lax
from jax.experimental import pallas as pl
from jax.experimental.pallas import tpu as pltpu
from jax.experimental.pallas import tpu_sc as plsc

_NC = 2
_NS = 16
_NW = _NC * _NS


def _ssp(h):
    return jnp.maximum(h, 0.0) + jnp.log(0.5 * (1.0 + jnp.exp(-jnp.abs(h))))


def _in2f_body(x_ref, win_ref, n_ref, y_ref, g_ref):
    y_ref[0] = jnp.dot(x_ref[0], win_ref[...], preferred_element_type=jnp.float32)
    g_ref[0] = n_ref[0] + pl.program_id(0) * x_ref.shape[1]


def _in2f(x, Win, neighbors):
    B, A, NIN = x.shape
    NF = Win.shape[1]
    NBH = neighbors.shape[2]
    return pl.pallas_call(
        _in2f_body,
        grid=(B,),
        in_specs=[
            pl.BlockSpec((1, A, NIN), lambda b: (b, 0, 0)),
            pl.BlockSpec((NIN, NF), lambda b: (0, 0)),
            pl.BlockSpec((1, A, NBH), lambda b: (b, 0, 0)),
        ],
        out_specs=[
            pl.BlockSpec((1, A, NF), lambda b: (b, 0, 0)),
            pl.BlockSpec((1, A, NBH), lambda b: (b, 0, 0)),
        ],
        out_shape=[
            jax.ShapeDtypeStruct((B, A, NF), jnp.float32),
            jax.ShapeDtypeStruct((B, A, NBH), jnp.int32),
        ],
    )(x, Win, neighbors)


def _sc_gather(y2d, gidx_flat):
    P = gidx_flat.shape[0]
    D = y2d.shape[1]
    rows_w = P // _NW
    K = 80
    steps = rows_w // K
    mesh = plsc.VectorSubcoreMesh(core_axis_name="c", subcore_axis_name="s")

    @functools.partial(
        pl.kernel,
        out_type=jax.ShapeDtypeStruct((P, D), jnp.float32),
        mesh=mesh,
        scratch_types=[
            pltpu.VMEM((K,), jnp.int32),
            pltpu.VMEM((K, D), jnp.float32),
            pltpu.SemaphoreType.DMA,
        ],
    )
    def k(y_hbm, nbr_hbm, out_hbm, idx_v, rows_v, sem):
        wid = lax.axis_index("s") * _NC + lax.axis_index("c")
        base = wid * rows_w

        def body(j, carry):
            off = base + j * K
            pltpu.sync_copy(nbr_hbm.at[pl.ds(off, K)], idx_v)
            pltpu.async_copy(y_hbm.at[idx_v], rows_v, sem).wait()
            pltpu.sync_copy(rows_v, out_hbm.at[pl.ds(off, K)])
            return carry

        lax.fori_loop(0, steps, body, 0)

    return k(y2d, gidx_flat)


def _fr_body(f_ref, yg_ref, m_ref, w1_ref, b1_ref, w2_ref, b2_ref, o_ref):
    TA, NBH, NG = f_ref.shape[1], f_ref.shape[2], f_ref.shape[3]
    NF = w2_ref.shape[1]
    f = f_ref[0].reshape(TA * NBH, NG)
    h = jnp.dot(f, w1_ref[...], preferred_element_type=jnp.float32) + b1_ref[0]
    h = _ssp(h)
    w = jnp.dot(h, w2_ref[...], preferred_element_type=jnp.float32) + b2_ref[0]
    p = w * yg_ref[0] * m_ref[0]
    o_ref[0] = jnp.sum(p.reshape(TA, NBH, NF), axis=1)


def _filter_reduce(f_ij, yg3, mask3, W1, b1, W2, b2, TA=40):
    B, A, NBH, NG = f_ij.shape
    NF = W2.shape[1]
    return pl.pallas_call(
        _fr_body,
        grid=(B, A // TA),
        in_specs=[
            pl.BlockSpec((1, TA, NBH, NG), lambda b, i: (b, i, 0, 0)),
            pl.BlockSpec((1, TA * NBH, NF), lambda b, i: (b, i, 0)),
            pl.BlockSpec((1, TA * NBH, 1), lambda b, i: (b, i, 0)),
            pl.BlockSpec((NG, NF), lambda b, i: (0, 0)),
            pl.BlockSpec((1, NF), lambda b, i: (0, 0)),
            pl.BlockSpec((NF, NF), lambda b, i: (0, 0)),
            pl.BlockSpec((1, NF), lambda b, i: (0, 0)),
        ],
        out_specs=pl.BlockSpec((1, TA, NF), lambda b, i: (b, i, 0)),
        out_shape=jax.ShapeDtypeStruct((B, A, NF), jnp.float32),
    )(f_ij, yg3, mask3, W1, b1.reshape(1, NF), W2, b2.reshape(1, NF))


def kernel(x, r_ij, neighbors, pairwise_mask, f_ij, Win, W1, b1, W2, b2):
    B, A, NBH = neighbors.shape
    NF = Win.shape[1]
    y, gidx = _in2f(x, Win, neighbors)
    yg = _sc_gather(y.reshape(B * A, NF), gidx.reshape(B * A * NBH))
    yg3 = yg.reshape(B, A * NBH, NF)
    mask3 = pairwise_mask.reshape(B, A * NBH, 1)
    return _filter_reduce(f_ij, yg3, mask3, W1, b1, W2, b2)
